# baseline (speedup 1.0000x reference)
import math

import jax
import jax.numpy as jnp
from jax import lax
from jax.experimental import pallas as pl

_END = 2
_LANES = 128


def _body(cur_ref, pcol_ref, ecol_ref, erow_ref, state_ref, dec_ref,
          outp_ref, outv_ref, oute_ref, outd_ref, outs_ref):
    beam, V = cur_ref.shape[1], cur_ref.shape[2]
    nch = V // _LANES
    L = _LANES
    nc = beam * beam
    x = cur_ref[0]
    pcol = pcol_ref[0]
    ecol = ecol_ref[0]
    erow = erow_ref[0]

    neg = jnp.float32(-jnp.inf)
    BIG = jnp.int32(1 << 30)
    vlog2 = int(math.log2(V))

    s = jnp.sum(jnp.exp(x), axis=-1, keepdims=True)
    c = pcol - jnp.log(s)

    x3 = x.reshape(beam, nch, L)
    ch = lax.broadcasted_iota(jnp.int32, (beam, nch, L), 1)

    m1 = jnp.max(x3, axis=1)
    a1 = jnp.min(jnp.where(x3 == m1[:, None, :], ch, BIG), axis=1)
    m2 = jnp.max(jnp.where(ch == a1[:, None, :], neg, x3), axis=1)
    a2 = jnp.min(jnp.where((x3 == m2[:, None, :]) & (ch != a1[:, None, :]),
                           ch, BIG), axis=1)

    bsub = lax.broadcasted_iota(jnp.int32, (beam, L), 0)
    lane = lax.broadcasted_iota(jnp.int32, (beam, L), 1)
    base = bsub * V + lane

    endm = ecol > 0
    e_lane = lane == (_END % L)
    e_flat = bsub * V + _END

    h1 = jnp.where(endm, jnp.where(e_lane, pcol, neg), m1 + c)
    f1 = jnp.where(endm, jnp.where(e_lane, e_flat, BIG), base + a1 * L)
    h2 = jnp.where(endm, neg, m2 + c)
    f2 = jnp.where(endm, BIG, base + a2 * L)

    t_one = jnp.zeros((beam, L), jnp.bool_)
    danger = jnp.full((beam, 1), neg)
    unsafe_v = jnp.zeros((beam, 1), jnp.bool_)
    lanek = lax.broadcasted_iota(jnp.int32, (beam, beam), 1)
    vals8 = jnp.zeros((beam, beam), jnp.float32)
    flats8 = jnp.zeros((beam, beam), jnp.int32)

    for k in range(beam):
        rm = jnp.max(h1, axis=1, keepdims=True)
        rf = jnp.min(jnp.where(h1 == rm, f1, BIG), axis=1, keepdims=True)
        unsafe_v = jnp.logical_or(
            unsafe_v, jnp.logical_and(rm <= danger, danger > neg))
        colm = f1 == rf
        second_pop = jnp.any(jnp.logical_and(colm, t_one), axis=1,
                             keepdims=True)
        danger = jnp.where(second_pop, jnp.maximum(danger, rm), danger)
        t_one = jnp.logical_or(t_one, colm)
        h1 = jnp.where(colm, h2, h1)
        f1 = jnp.where(colm, f2, f1)
        h2 = jnp.where(colm, neg, h2)
        f2 = jnp.where(colm, BIG, f2)
        vals8 = jnp.where(lanek == k, rm, vals8)
        flats8 = jnp.where(lanek == k, rf, flats8)

    unsafe = jnp.any(unsafe_v)

    v4a = vals8[:, :, None, None]
    v4b = vals8[None, None, :, :]
    f4a = flats8[:, :, None, None]
    f4b = flats8[None, None, :, :]
    gt = (v4a > v4b) | ((v4a == v4b) & (f4a < f4b))
    rank = jnp.sum(gt.astype(jnp.int32), axis=(0, 1))
    kidx3 = lax.broadcasted_iota(jnp.int32, (beam, beam, beam), 2)
    M = rank[:, :, None] == kidx3
    val_row = jnp.sum(jnp.where(M, vals8[:, :, None], 0.0),
                      axis=(0, 1))[None, :]
    fi_int = jnp.sum(jnp.where(M, flats8[:, :, None], 0),
                     axis=(0, 1))[None, :]
    voc_row = fi_int & (V - 1)
    N = jnp.sum(M.astype(jnp.float32), axis=1)
    eye = (lax.broadcasted_iota(jnp.int32, (beam, beam), 0)
           == lax.broadcasted_iota(jnp.int32, (beam, beam), 1)
           ).astype(jnp.float32)
    W = lax.dot_general(N, eye, (((0,), (0,)), ((), ())),
                        preferred_element_type=jnp.float32)

    outp_ref[0] = val_row
    outv_ref[0] = voc_row
    oute_ref[0] = ((erow > 0) | (voc_row == _END)).astype(jnp.int32)
    dec = dec_ref[0].astype(jnp.float32)
    outd_ref[0] = lax.dot_general(
        dec, W, (((1,), (1,)), ((), ())),
        preferred_element_type=jnp.float32).astype(jnp.int32)
    outs_ref[0] = lax.dot_general(
        W, state_ref[0], (((1,), (0,)), ((), ())),
        preferred_element_type=jnp.float32)

    l8 = lax.broadcasted_iota(jnp.int32, (1, beam), 1)
    sub8 = lax.broadcasted_iota(jnp.int32, (beam, beam), 0)

    @pl.when(unsafe)
    def _():
        col = lax.broadcasted_iota(jnp.int32, (beam, V), 1)
        bs2 = lax.broadcasted_iota(jnp.int32, (beam, V), 0)
        flat = bs2 * V + col
        total = jnp.where(endm, jnp.where(col == _END, pcol, neg), x + c)

        vr = jnp.zeros((1, beam), jnp.float32)
        vo = jnp.zeros((1, beam), jnp.int32)
        Ws = jnp.zeros((beam, beam), jnp.float32)
        tt = total
        for k in range(beam):
            mm = jnp.max(tt)
            ff = jnp.min(jnp.where(tt == mm, flat, BIG))
            tt = jnp.where(flat == ff, neg, tt)
            vv = ff & (V - 1)
            bb = lax.shift_right_logical(ff, vlog2)
            vr = jnp.where(l8 == k, mm, vr)
            vo = jnp.where(l8 == k, vv, vo)
            Ws = Ws + jnp.where((sub8 == k) & (lanek == bb), 1.0, 0.0)

        outp_ref[0] = vr
        outv_ref[0] = vo
        oute_ref[0] = ((erow > 0) | (vo == _END)).astype(jnp.int32)
        dec2 = dec_ref[0].astype(jnp.float32)
        outd_ref[0] = lax.dot_general(
            dec2, Ws, (((1,), (1,)), ((), ())),
            preferred_element_type=jnp.float32).astype(jnp.int32)
        outs_ref[0] = lax.dot_general(
            Ws, state_ref[0], (((1,), (0,)), ((), ())),
            preferred_element_type=jnp.float32)


def kernel(cur_proba, proba, is_ended, state, decodeds):
    batch, beam, V = cur_proba.shape
    d = state.shape[-1]
    t = decodeds.shape[0]
    pcol = proba.reshape(batch, beam, 1)
    ecol = is_ended.astype(jnp.int32).reshape(batch, beam, 1)
    erow = is_ended.astype(jnp.int32).reshape(batch, 1, beam)
    dec3 = decodeds.astype(jnp.int32).transpose(1, 0, 2)

    outs = pl.pallas_call(
        _body,
        grid=(batch,),
        in_specs=[
            pl.BlockSpec((1, beam, V), lambda b: (b, 0, 0)),
            pl.BlockSpec((1, beam, 1), lambda b: (b, 0, 0)),
            pl.BlockSpec((1, beam, 1), lambda b: (b, 0, 0)),
            pl.BlockSpec((1, 1, beam), lambda b: (b, 0, 0)),
            pl.BlockSpec((1, beam, d), lambda b: (b, 0, 0)),
            pl.BlockSpec((1, t, beam), lambda b: (b, 0, 0)),
        ],
        out_specs=[
            pl.BlockSpec((1, 1, beam), lambda b: (b, 0, 0)),
            pl.BlockSpec((1, 1, beam), lambda b: (b, 0, 0)),
            pl.BlockSpec((1, 1, beam), lambda b: (b, 0, 0)),
            pl.BlockSpec((1, t, beam), lambda b: (b, 0, 0)),
            pl.BlockSpec((1, beam, d), lambda b: (b, 0, 0)),
        ],
        out_shape=[
            jax.ShapeDtypeStruct((batch, 1, beam), jnp.float32),
            jax.ShapeDtypeStruct((batch, 1, beam), jnp.int32),
            jax.ShapeDtypeStruct((batch, 1, beam), jnp.int32),
            jax.ShapeDtypeStruct((batch, t, beam), jnp.int32),
            jax.ShapeDtypeStruct((batch, beam, d), jnp.float32),
        ],
    )(cur_proba, pcol, ecol, erow, state, dec3)

    new_proba3, voc3, ended3, decg3, new_state = outs
    new_proba = new_proba3.reshape(batch, beam)
    topk_voc = voc3.reshape(batch, beam)
    new_is_ended = ended3.reshape(batch, beam).astype(bool)
    gathered_dec = decg3.transpose(1, 0, 2)
    new_decodeds = jnp.concatenate([gathered_dec, topk_voc[None]], axis=0)
    cur_input = topk_voc.reshape(-1)
    return new_proba, new_decodeds, new_is_ended, new_state, cur_input


# speedup vs baseline: 1.9379x; 1.9379x over previous
import math

import jax
import jax.numpy as jnp
from jax import lax
from jax.experimental import pallas as pl
from jax.experimental.pallas import tpu as pltpu
from jax.experimental.pallas import tpu_sc as plsc
import functools


def _sc_state_gather(table, gidx):
    B, D = table.shape
    info = plsc.get_sparse_core_info()
    NC, NS = info.num_cores, info.num_subcores
    NW = NC * NS
    b_per_w = B // NW
    mesh = plsc.VectorSubcoreMesh(core_axis_name="c", subcore_axis_name="s")

    @functools.partial(
        pl.kernel, mesh=mesh,
        out_type=jax.ShapeDtypeStruct((B, D), jnp.float32),
        scratch_types=[
            pltpu.VMEM((b_per_w,), jnp.int32),
            pltpu.VMEM((b_per_w, D), jnp.float32),
            pltpu.SemaphoreType.DMA,
        ],
    )
    def gather_rows(table_hbm, idx_hbm, out_hbm, idx_v, rows_v, sem):
        wid = lax.axis_index("s") * NC + lax.axis_index("c")
        base = wid * b_per_w
        pltpu.sync_copy(idx_hbm.at[pl.ds(base, b_per_w)], idx_v)
        pltpu.async_copy(table_hbm.at[idx_v], rows_v, sem).wait()
        pltpu.sync_copy(rows_v, out_hbm.at[pl.ds(base, b_per_w)])

    return gather_rows(table, gidx)

_END = 2
_LANES = 128


def _body(cur_ref, pcol_ref, ecol_ref, erow_ref, dec_ref,
          outp_ref, outv_ref, oute_ref, outd_ref, outg_ref,
          m_s, fi_s):
    beam, V = cur_ref.shape[1], cur_ref.shape[2]
    nchunk = V // _LANES
    x = cur_ref[0]
    pcol = pcol_ref[0]
    ecol = ecol_ref[0]
    erow = erow_ref[0]
    dec = dec_ref[0].astype(jnp.float32)
    prog = pl.program_id(0)

    neg = jnp.float32(-jnp.inf)
    BIG = jnp.int32(1 << 30)

    s = jnp.sum(jnp.exp(x), axis=-1, keepdims=True)
    c = pcol - jnp.log(s)

    x3 = x.reshape(beam, nchunk, _LANES)
    ch = lax.broadcasted_iota(jnp.int32, (beam, nchunk, _LANES), 1)

    m1 = jnp.max(x3, axis=1)
    a1 = jnp.min(jnp.where(x3 == m1[:, None, :], ch, BIG), axis=1)
    x3m = jnp.where(ch == a1[:, None, :], neg, x3)
    m2 = jnp.max(x3m, axis=1)
    a2 = jnp.min(jnp.where(x3m == m2[:, None, :], ch, BIG), axis=1)

    bsub = lax.broadcasted_iota(jnp.int32, (beam, _LANES), 0)
    lane = lax.broadcasted_iota(jnp.int32, (beam, _LANES), 1)
    base = bsub * V + lane
    h1 = m1 + c
    f1 = base + a1 * _LANES
    h2 = m2 + c
    f2 = base + a2 * _LANES

    endm = ecol > 0
    e_lane = lane == (_END % _LANES)
    e_flat = bsub * V + _END
    h1 = jnp.where(endm, jnp.where(e_lane, pcol, neg), h1)
    f1 = jnp.where(endm, jnp.where(e_lane, e_flat, BIG), f1)
    h2 = jnp.where(endm, neg, h2)
    f2 = jnp.where(endm, BIG, f2)

    t_cnt = jnp.zeros((beam, _LANES), jnp.int32)
    danger = neg
    deg = jnp.bool_(False)

    l8 = lax.broadcasted_iota(jnp.int32, (1, beam), 1)
    sub8 = lax.broadcasted_iota(jnp.int32, (beam, beam), 0)
    lan8 = lax.broadcasted_iota(jnp.int32, (beam, beam), 1)
    val_row = jnp.zeros((1, beam), jnp.float32)
    voc_row = jnp.zeros((1, beam), jnp.int32)
    g_row = jnp.zeros((1, beam), jnp.int32)
    W = jnp.zeros((beam, beam), jnp.float32)
    vlog2 = int(math.log2(V))
    removed = []

    for k in range(beam):
        m_fast = jnp.max(h1)
        safe = jnp.logical_and(jnp.logical_not(deg), m_fast > danger)

        @pl.when(safe)
        def _(h1=h1, f1=f1, m_fast=m_fast):
            fi_f = jnp.min(jnp.where(h1 == m_fast, f1, BIG))
            m_s[0] = m_fast
            fi_s[0] = fi_f

        @pl.when(jnp.logical_not(safe))
        def _(removed=tuple(removed)):
            f3 = lax.broadcasted_iota(jnp.int32, (beam, nchunk, _LANES), 0) * V \
                + lax.broadcasted_iota(jnp.int32, (beam, nchunk, _LANES), 1) * _LANES \
                + lax.broadcasted_iota(jnp.int32, (beam, nchunk, _LANES), 2)
            t3 = x3 + c[:, None]
            e3 = endm[:, :, None]
            bflat = lax.broadcasted_iota(jnp.int32, (beam, 1, 1), 0) * V + _END
            t3 = jnp.where(e3, jnp.where(f3 == bflat, pcol[:, None], neg), t3)
            rm = jnp.zeros((beam, nchunk, _LANES), jnp.bool_)
            for r in removed:
                rm = jnp.logical_or(rm, f3 == r)
            t3 = jnp.where(rm, neg, t3)
            m_slow = jnp.max(t3)
            fi_slow = jnp.min(jnp.where(t3 == m_slow, f3, BIG))
            m_s[0] = m_slow
            fi_s[0] = fi_slow

        m = m_s[0]
        fi = fi_s[0]
        removed.append(fi)

        colm = f1 == fi
        second_pop = jnp.any(jnp.logical_and(colm, t_cnt == 1))
        danger = jnp.where(jnp.logical_and(safe, second_pop),
                           jnp.maximum(danger, m), danger)
        t_cnt = t_cnt + colm.astype(jnp.int32)
        h1 = jnp.where(colm, h2, h1)
        f1 = jnp.where(colm, f2, f1)
        h2 = jnp.where(colm, neg, h2)
        f2 = jnp.where(colm, BIG, f2)
        deg = jnp.logical_or(deg, jnp.logical_not(safe))

        vk = fi & (V - 1) if (1 << vlog2) == V else fi % V
        bk = lax.shift_right_logical(fi, vlog2) if (1 << vlog2) == V else fi // V
        val_row = jnp.where(l8 == k, m, val_row)
        voc_row = jnp.where(l8 == k, vk, voc_row)
        g_row = jnp.where(l8 == k, prog * beam + bk, g_row)
        W = W + jnp.where((sub8 == k) & (lan8 == bk), 1.0, 0.0)

    outp_ref[0] = val_row
    outv_ref[0] = voc_row
    oute_ref[0] = ((erow > 0) | (voc_row == _END)).astype(jnp.int32)
    gathered = lax.dot_general(dec, W, (((1,), (1,)), ((), ())),
                               preferred_element_type=jnp.float32)
    outd_ref[0] = gathered.astype(jnp.int32)
    outg_ref[0] = g_row


def kernel(cur_proba, proba, is_ended, state, decodeds):
    batch, beam, V = cur_proba.shape
    d = state.shape[-1]
    t = decodeds.shape[0]
    pcol = proba.reshape(batch, beam, 1)
    ecol = is_ended.astype(jnp.int32).reshape(batch, beam, 1)
    erow = is_ended.astype(jnp.int32).reshape(batch, 1, beam)
    dec3 = decodeds.astype(jnp.int32).transpose(1, 0, 2)

    outs = pl.pallas_call(
        _body,
        grid=(batch,),
        in_specs=[
            pl.BlockSpec((1, beam, V), lambda b: (b, 0, 0)),
            pl.BlockSpec((1, beam, 1), lambda b: (b, 0, 0)),
            pl.BlockSpec((1, beam, 1), lambda b: (b, 0, 0)),
            pl.BlockSpec((1, 1, beam), lambda b: (b, 0, 0)),
            pl.BlockSpec((1, t, beam), lambda b: (b, 0, 0)),
        ],
        out_specs=[
            pl.BlockSpec((1, 1, beam), lambda b: (b, 0, 0)),
            pl.BlockSpec((1, 1, beam), lambda b: (b, 0, 0)),
            pl.BlockSpec((1, 1, beam), lambda b: (b, 0, 0)),
            pl.BlockSpec((1, t, beam), lambda b: (b, 0, 0)),
            pl.BlockSpec((1, 1, beam), lambda b: (b, 0, 0)),
        ],
        out_shape=[
            jax.ShapeDtypeStruct((batch, 1, beam), jnp.float32),
            jax.ShapeDtypeStruct((batch, 1, beam), jnp.int32),
            jax.ShapeDtypeStruct((batch, 1, beam), jnp.int32),
            jax.ShapeDtypeStruct((batch, t, beam), jnp.int32),
            jax.ShapeDtypeStruct((batch, 1, beam), jnp.int32),
        ],
        scratch_shapes=[
            pltpu.SMEM((1,), jnp.float32),
            pltpu.SMEM((1,), jnp.int32),
        ],
    )(cur_proba, pcol, ecol, erow, dec3)

    new_proba3, voc3, ended3, decg3, gidx3 = outs
    new_state = _sc_state_gather(state.reshape(batch * beam, d),
                                 gidx3.reshape(batch * beam)
                                 ).reshape(batch, beam, d)
    new_proba = new_proba3.reshape(batch, beam)
    topk_voc = voc3.reshape(batch, beam)
    new_is_ended = ended3.reshape(batch, beam).astype(bool)
    gathered_dec = decg3.transpose(1, 0, 2)
    new_decodeds = jnp.concatenate([gathered_dec, topk_voc[None]], axis=0)
    cur_input = topk_voc.reshape(-1)
    return new_proba, new_decodeds, new_is_ended, new_state, cur_input
